# Initial kernel scaffold; baseline (speedup 1.0000x reference)
import jax
import jax.numpy as jnp
from jax.experimental import pallas as pl


def kernel(Z, hyperedge_index, num_nodes, hyperedge_weight):
    raise NotImplementedError("write your pallas kernel here")


# speedup vs baseline: 107.3025x; 107.3025x over previous
import functools

import jax
import jax.numpy as jnp
from jax import lax
from jax.experimental import pallas as pl
from jax.experimental.pallas import tpu as pltpu
from jax.experimental.pallas import tpu_sc as plsc

N_CORES = 2
N_SUBCORES = 16
N_WORKERS = N_CORES * N_SUBCORES
N_PAD = 100096
SLICE = N_PAD // N_SUBCORES
CH = 2000
K = 8
RB = 3128


def _degrees_body(hi_hbm, w_hbm, z1_hbm, ones_hbm, dv_out, de_out,
                  idx_n, idx_e, wv, ones_v, w_sp, dv_sp, de_sp, sem):
    cid = lax.axis_index("c")
    sid = lax.axis_index("s")
    wid = cid * N_SUBCORES + sid
    soff = sid * SLICE
    pltpu.sync_copy(w_hbm.at[pl.ds(soff, SLICE)], w_sp.at[pl.ds(soff, SLICE)])
    pltpu.sync_copy(z1_hbm, dv_sp.at[pl.ds(soff, SLICE)])
    pltpu.sync_copy(z1_hbm, de_sp.at[pl.ds(soff, SLICE)])
    pltpu.sync_copy(ones_hbm, ones_v)
    plsc.subcore_barrier()

    n_pairs = hi_hbm.shape[0] // 2
    per_w = n_pairs // N_WORKERS
    base = wid * per_w

    def body(i, carry):
        off = base + i * CH
        pltpu.sync_copy(hi_hbm.at[pl.ds(off, CH)], idx_n)
        pltpu.sync_copy(hi_hbm.at[pl.ds(n_pairs + off, CH)], idx_e)
        pltpu.async_copy(w_sp.at[idx_e], wv, sem).wait()
        pltpu.sync_copy(wv, dv_sp.at[idx_n], add=True)
        pltpu.sync_copy(ones_v, de_sp.at[idx_e], add=True)
        return carry

    lax.fori_loop(0, per_w // CH, body, 0)
    plsc.subcore_barrier()
    pltpu.sync_copy(dv_sp.at[pl.ds(soff, SLICE)],
                    dv_out.at[cid, pl.ds(soff, SLICE)])
    pltpu.sync_copy(de_sp.at[pl.ds(soff, SLICE)],
                    de_out.at[cid, pl.ds(soff, SLICE)])


def _edge_sums_body(hi_hbm, nz_hbm, z8_hbm, s_out,
                    idx_n, idx_e, rows_v, nz_sp, s_sp, sem):
    cid = lax.axis_index("c")
    sid = lax.axis_index("s")
    wid = cid * N_SUBCORES + sid
    soff = sid * SLICE
    pltpu.sync_copy(nz_hbm.at[pl.ds(soff, SLICE)],
                    nz_sp.at[pl.ds(soff, SLICE)])
    pltpu.sync_copy(z8_hbm, s_sp.at[pl.ds(soff, SLICE)])
    plsc.subcore_barrier()

    n_pairs = hi_hbm.shape[0] // 2
    per_w = n_pairs // N_WORKERS
    base = wid * per_w

    def body(i, carry):
        off = base + i * CH
        pltpu.sync_copy(hi_hbm.at[pl.ds(off, CH)], idx_n)
        pltpu.sync_copy(hi_hbm.at[pl.ds(n_pairs + off, CH)], idx_e)
        pltpu.async_copy(nz_sp.at[idx_n], rows_v, sem).wait()
        pltpu.sync_copy(rows_v, s_sp.at[idx_e], add=True)
        return carry

    lax.fori_loop(0, per_w // CH, body, 0)
    plsc.subcore_barrier()
    pltpu.sync_copy(s_sp.at[pl.ds(soff, SLICE)],
                    s_out.at[cid, pl.ds(soff, SLICE)])


def _normalize_body(dvp_ref, z_ref, nz_ref, f_ref):
    dv = dvp_ref[0] + dvp_ref[1]
    dv = jnp.where(dv == 0.0, 1.0, dv)
    z = z_ref[...]
    nz_ref[...] = lax.rsqrt(dv) * z

    @pl.when(pl.program_id(0) == 0)
    def _():
        f_ref[...] = jnp.zeros_like(f_ref)

    f_ref[...] += jnp.sum(z * z * dv, axis=0, keepdims=True)


def _loss_body(sp_ref, dep_ref, w_ref, f_ref, loss_ref, th_ref):
    i = pl.program_id(0)
    s = sp_ref[0] + sp_ref[1]
    de = dep_ref[0] + dep_ref[1]
    de = jnp.where(de == 0.0, 1.0, de)
    w = w_ref[...]

    @pl.when(i == 0)
    def _():
        th_ref[...] = jnp.zeros_like(th_ref)

    th_ref[...] += jnp.sum(w * s * s / de, axis=0, keepdims=True)

    @pl.when(i == pl.num_programs(0) - 1)
    def _():
        rq = 1.0 - th_ref[...] / (f_ref[...] + 1e-8)
        loss_ref[...] = jnp.mean(rq, axis=(0, 1), keepdims=True)


def kernel(Z, hyperedge_index, num_nodes, hyperedge_weight):
    del num_nodes
    n = Z.shape[0]
    f32 = jnp.float32

    hi_flat = jnp.reshape(hyperedge_index, (-1,))
    z_pad = jnp.pad(Z.astype(f32), ((0, N_PAD - n), (0, 0)))
    w_pad = jnp.pad(hyperedge_weight.astype(f32), (0, N_PAD - n))
    w_pad2 = w_pad[:, None]
    zeros1 = jnp.zeros((SLICE,), f32)
    zeros8 = jnp.zeros((SLICE, K), f32)
    ones_ch = jnp.ones((CH,), f32)

    mesh = plsc.VectorSubcoreMesh(core_axis_name="c", subcore_axis_name="s")
    sc_params = pltpu.CompilerParams(use_tc_tiling_on_sc=False)

    degrees = pl.kernel(
        _degrees_body,
        out_type=(
            jax.ShapeDtypeStruct((N_CORES, N_PAD), f32),
            jax.ShapeDtypeStruct((N_CORES, N_PAD), f32),
        ),
        mesh=mesh,
        scratch_types=(
            pltpu.VMEM((CH,), jnp.int32),
            pltpu.VMEM((CH,), jnp.int32),
            pltpu.VMEM((CH,), f32),
            pltpu.VMEM((CH,), f32),
            pltpu.VMEM_SHARED((N_PAD,), f32),
            pltpu.VMEM_SHARED((N_PAD,), f32),
            pltpu.VMEM_SHARED((N_PAD,), f32),
            pltpu.SemaphoreType.DMA,
        ),
        compiler_params=sc_params,
        name="hg_degrees_sc",
    )
    dv_p, de_p = degrees(hi_flat, w_pad, zeros1, ones_ch)
    dv_p3 = jnp.reshape(dv_p, (N_CORES, N_PAD, 1))
    de_p3 = jnp.reshape(de_p, (N_CORES, N_PAD, 1))

    grid = N_PAD // RB
    nz_pad, f_dv_f = pl.pallas_call(
        _normalize_body,
        grid=(grid,),
        in_specs=[
            pl.BlockSpec((N_CORES, RB, 1), lambda i: (0, i, 0)),
            pl.BlockSpec((RB, K), lambda i: (i, 0)),
        ],
        out_specs=[
            pl.BlockSpec((RB, K), lambda i: (i, 0)),
            pl.BlockSpec((1, K), lambda i: (0, 0)),
        ],
        out_shape=[
            jax.ShapeDtypeStruct((N_PAD, K), f32),
            jax.ShapeDtypeStruct((1, K), f32),
        ],
        name="hg_normalize_tc",
    )(dv_p3, z_pad)

    edge_sums = pl.kernel(
        _edge_sums_body,
        out_type=jax.ShapeDtypeStruct((N_CORES, N_PAD, K), f32),
        mesh=mesh,
        scratch_types=(
            pltpu.VMEM((CH,), jnp.int32),
            pltpu.VMEM((CH,), jnp.int32),
            pltpu.VMEM((CH, K), f32),
            pltpu.VMEM_SHARED((N_PAD, K), f32),
            pltpu.VMEM_SHARED((N_PAD, K), f32),
            pltpu.SemaphoreType.DMA,
        ),
        compiler_params=sc_params,
        name="hg_edge_sums_sc",
    )
    s_p = edge_sums(hi_flat, nz_pad, zeros8)

    loss2d = pl.pallas_call(
        _loss_body,
        grid=(grid,),
        in_specs=[
            pl.BlockSpec((N_CORES, RB, K), lambda i: (0, i, 0)),
            pl.BlockSpec((N_CORES, RB, 1), lambda i: (0, i, 0)),
            pl.BlockSpec((RB, 1), lambda i: (i, 0)),
            pl.BlockSpec((1, K), lambda i: (0, 0)),
        ],
        out_specs=pl.BlockSpec((1, 1), lambda i: (0, 0)),
        out_shape=jax.ShapeDtypeStruct((1, 1), f32),
        scratch_shapes=[pltpu.VMEM((1, K), f32)],
        name="hg_loss_tc",
    )(s_p, de_p3, w_pad2, f_dv_f)

    return jnp.reshape(loss2d, ())


# speedup vs baseline: 117.8048x; 1.0979x over previous
import jax
import jax.numpy as jnp
from jax import lax
from jax.experimental import pallas as pl
from jax.experimental.pallas import tpu as pltpu
from jax.experimental.pallas import tpu_sc as plsc

N_CORES = 2
N_SUBCORES = 16
N_WORKERS = N_CORES * N_SUBCORES
N_PAD = 100096
SLICE = N_PAD // N_SUBCORES
CH_A = 4000
CH_C = 1000
K = 8
N_ROWS = 100000
RB = 5000


def _degrees_body(hi, w_hbm, z1, ones_hbm, dv_out, de_out,
                  in_a, ie_a, in_b, ie_b, wv_a, wv_b, ones_v,
                  w_sp, dv_sp, de_sp,
                  sem_la, sem_lb, sem_g, sem_sa, sem_sb):
    cid = lax.axis_index("c")
    sid = lax.axis_index("s")
    wid = cid * N_SUBCORES + sid
    soff = sid * SLICE
    pltpu.sync_copy(w_hbm.at[pl.ds(soff, SLICE)], w_sp.at[pl.ds(soff, SLICE)])
    pltpu.sync_copy(z1, dv_sp.at[pl.ds(soff, SLICE)])
    pltpu.sync_copy(z1, de_sp.at[pl.ds(soff, SLICE)])
    pltpu.sync_copy(ones_hbm, ones_v)
    plsc.subcore_barrier()

    n_pairs = hi.shape[0] // 2
    per_w = n_pairs // N_WORKERS
    nit2 = per_w // (2 * CH_A)
    base = wid * per_w

    def loads(i, idx_n, idx_e, sem):
        pltpu.async_copy(hi.at[pl.ds(base + i * CH_A, CH_A)], idx_n, sem)
        pltpu.async_copy(hi.at[pl.ds(n_pairs + base + i * CH_A, CH_A)], idx_e, sem)

    def wait_loads(idx_n, idx_e, sem):
        pltpu.make_async_copy(hi.at[pl.ds(base, CH_A)], idx_n, sem).wait()
        pltpu.make_async_copy(hi.at[pl.ds(base, CH_A)], idx_e, sem).wait()

    def scatters(idx_n, idx_e, wv, sem):
        pltpu.async_copy(wv, dv_sp.at[idx_n], sem, add=True)
        pltpu.async_copy(ones_v, de_sp.at[idx_e], sem, add=True)

    def wait_scatters(idx_n, idx_e, wv, sem):
        pltpu.make_async_copy(wv, dv_sp.at[idx_n], sem).wait()
        pltpu.make_async_copy(ones_v, de_sp.at[idx_e], sem).wait()

    loads(0, in_a, ie_a, sem_la)

    def body(j, carry):
        @pl.when(j > 0)
        def _():
            wait_scatters(in_b, ie_b, wv_b, sem_sb)

        wait_loads(in_a, ie_a, sem_la)
        ga = pltpu.async_copy(w_sp.at[ie_a], wv_a, sem_g)
        loads(2 * j + 1, in_b, ie_b, sem_lb)
        ga.wait()
        scatters(in_a, ie_a, wv_a, sem_sa)
        wait_loads(in_b, ie_b, sem_lb)
        pltpu.async_copy(w_sp.at[ie_b], wv_b, sem_g).wait()
        wait_scatters(in_a, ie_a, wv_a, sem_sa)

        @pl.when(j < nit2 - 1)
        def _():
            loads(2 * j + 2, in_a, ie_a, sem_la)

        scatters(in_b, ie_b, wv_b, sem_sb)
        return carry

    lax.fori_loop(0, nit2, body, 0)
    wait_scatters(in_b, ie_b, wv_b, sem_sb)
    plsc.subcore_barrier()
    pltpu.sync_copy(dv_sp.at[pl.ds(soff, SLICE)],
                    dv_out.at[cid, pl.ds(soff, SLICE)])
    pltpu.sync_copy(de_sp.at[pl.ds(soff, SLICE)],
                    de_out.at[cid, pl.ds(soff, SLICE)])


def _edge_sums_body(hi, nz_hbm, z8_hbm, s_out,
                    in_a, ie_a, in_b, ie_b, rows_a, rows_b,
                    nz_sp, s_sp,
                    sem_la, sem_lb, sem_g, sem_sa, sem_sb):
    cid = lax.axis_index("c")
    sid = lax.axis_index("s")
    wid = cid * N_SUBCORES + sid
    soff = sid * SLICE
    pltpu.sync_copy(nz_hbm.at[pl.ds(soff, SLICE)],
                    nz_sp.at[pl.ds(soff, SLICE)])
    pltpu.sync_copy(z8_hbm, s_sp.at[pl.ds(soff, SLICE)])
    plsc.subcore_barrier()

    n_pairs = hi.shape[0] // 2
    per_w = n_pairs // N_WORKERS
    nit2 = per_w // (2 * CH_C)
    base = wid * per_w

    def loads(i, idx_n, idx_e, sem):
        pltpu.async_copy(hi.at[pl.ds(base + i * CH_C, CH_C)], idx_n, sem)
        pltpu.async_copy(hi.at[pl.ds(n_pairs + base + i * CH_C, CH_C)], idx_e, sem)

    def wait_loads(idx_n, idx_e, sem):
        pltpu.make_async_copy(hi.at[pl.ds(base, CH_C)], idx_n, sem).wait()
        pltpu.make_async_copy(hi.at[pl.ds(base, CH_C)], idx_e, sem).wait()

    loads(0, in_a, ie_a, sem_la)

    def body(j, carry):
        @pl.when(j > 0)
        def _():
            pltpu.make_async_copy(rows_b, s_sp.at[ie_b], sem_sb).wait()

        wait_loads(in_a, ie_a, sem_la)
        ga = pltpu.async_copy(nz_sp.at[in_a], rows_a, sem_g)
        loads(2 * j + 1, in_b, ie_b, sem_lb)
        ga.wait()
        pltpu.async_copy(rows_a, s_sp.at[ie_a], sem_sa, add=True)
        wait_loads(in_b, ie_b, sem_lb)
        pltpu.async_copy(nz_sp.at[in_b], rows_b, sem_g).wait()
        pltpu.make_async_copy(rows_a, s_sp.at[ie_a], sem_sa).wait()

        @pl.when(j < nit2 - 1)
        def _():
            loads(2 * j + 2, in_a, ie_a, sem_la)

        pltpu.async_copy(rows_b, s_sp.at[ie_b], sem_sb, add=True)
        return carry

    lax.fori_loop(0, nit2, body, 0)
    pltpu.make_async_copy(rows_b, s_sp.at[ie_b], sem_sb).wait()
    plsc.subcore_barrier()
    pltpu.sync_copy(s_sp.at[pl.ds(soff, SLICE)],
                    s_out.at[cid, pl.ds(soff, SLICE)])


def _normalize_body(dvp_ref, z_ref, nz_ref, f_ref):
    dv = dvp_ref[0] + dvp_ref[1]
    dv = jnp.where(dv == 0.0, 1.0, dv)
    z = z_ref[...]
    nz_ref[...] = lax.rsqrt(dv) * z

    @pl.when(pl.program_id(0) == 0)
    def _():
        f_ref[...] = jnp.zeros_like(f_ref)

    f_ref[...] += jnp.sum(z * z * dv, axis=0, keepdims=True)


def _loss_body(sp_ref, dep_ref, w_ref, f_ref, loss_ref, th_ref):
    i = pl.program_id(0)
    s = sp_ref[0] + sp_ref[1]
    de = dep_ref[0] + dep_ref[1]
    de = jnp.where(de == 0.0, 1.0, de)
    w = w_ref[...]

    @pl.when(i == 0)
    def _():
        th_ref[...] = jnp.zeros_like(th_ref)

    th_ref[...] += jnp.sum(w * s * s / de, axis=0, keepdims=True)

    @pl.when(i == pl.num_programs(0) - 1)
    def _():
        rq = 1.0 - th_ref[...] / (f_ref[...] + 1e-8)
        loss_ref[...] = jnp.mean(rq, axis=(0, 1), keepdims=True)


def kernel(Z, hyperedge_index, num_nodes, hyperedge_weight):
    del num_nodes
    n = Z.shape[0]
    f32 = jnp.float32

    hi_flat = jnp.reshape(hyperedge_index, (-1,))
    w_pad = jnp.pad(hyperedge_weight.astype(f32), (0, N_PAD - n))
    w_pad2 = w_pad[:, None]
    zeros1 = jnp.zeros((SLICE,), f32)
    zeros8 = jnp.zeros((SLICE, K), f32)
    ones_ch = jnp.ones((CH_A,), f32)

    mesh = plsc.VectorSubcoreMesh(core_axis_name="c", subcore_axis_name="s")
    sc_params = pltpu.CompilerParams(use_tc_tiling_on_sc=False)

    degrees = pl.kernel(
        _degrees_body,
        out_type=(
            jax.ShapeDtypeStruct((N_CORES, N_PAD), f32),
            jax.ShapeDtypeStruct((N_CORES, N_PAD), f32),
        ),
        mesh=mesh,
        scratch_types=(
            pltpu.VMEM((CH_A,), jnp.int32),
            pltpu.VMEM((CH_A,), jnp.int32),
            pltpu.VMEM((CH_A,), jnp.int32),
            pltpu.VMEM((CH_A,), jnp.int32),
            pltpu.VMEM((CH_A,), f32),
            pltpu.VMEM((CH_A,), f32),
            pltpu.VMEM((CH_A,), f32),
            pltpu.VMEM_SHARED((N_PAD,), f32),
            pltpu.VMEM_SHARED((N_PAD,), f32),
            pltpu.VMEM_SHARED((N_PAD,), f32),
            pltpu.SemaphoreType.DMA,
            pltpu.SemaphoreType.DMA,
            pltpu.SemaphoreType.DMA,
            pltpu.SemaphoreType.DMA,
            pltpu.SemaphoreType.DMA,
        ),
        compiler_params=sc_params,
        name="hg_degrees_sc",
    )
    dv_p, de_p = degrees(hi_flat, w_pad, zeros1, ones_ch)
    dv_p3 = jnp.reshape(dv_p, (N_CORES, N_PAD, 1))
    de_p3 = jnp.reshape(de_p, (N_CORES, N_PAD, 1))

    grid = N_ROWS // RB
    nz_pad, f_dv_f = pl.pallas_call(
        _normalize_body,
        grid=(grid,),
        in_specs=[
            pl.BlockSpec((N_CORES, RB, 1), lambda i: (0, i, 0)),
            pl.BlockSpec((RB, K), lambda i: (i, 0)),
        ],
        out_specs=[
            pl.BlockSpec((RB, K), lambda i: (i, 0)),
            pl.BlockSpec((1, K), lambda i: (0, 0)),
        ],
        out_shape=[
            jax.ShapeDtypeStruct((N_PAD, K), f32),
            jax.ShapeDtypeStruct((1, K), f32),
        ],
        name="hg_normalize_tc",
    )(dv_p3, Z.astype(f32))

    edge_sums = pl.kernel(
        _edge_sums_body,
        out_type=jax.ShapeDtypeStruct((N_CORES, N_PAD, K), f32),
        mesh=mesh,
        scratch_types=(
            pltpu.VMEM((CH_C,), jnp.int32),
            pltpu.VMEM((CH_C,), jnp.int32),
            pltpu.VMEM((CH_C,), jnp.int32),
            pltpu.VMEM((CH_C,), jnp.int32),
            pltpu.VMEM((CH_C, K), f32),
            pltpu.VMEM((CH_C, K), f32),
            pltpu.VMEM_SHARED((N_PAD, K), f32),
            pltpu.VMEM_SHARED((N_PAD, K), f32),
            pltpu.SemaphoreType.DMA,
            pltpu.SemaphoreType.DMA,
            pltpu.SemaphoreType.DMA,
            pltpu.SemaphoreType.DMA,
            pltpu.SemaphoreType.DMA,
        ),
        compiler_params=sc_params,
        name="hg_edge_sums_sc",
    )
    s_p = edge_sums(hi_flat, nz_pad, zeros8)

    loss2d = pl.pallas_call(
        _loss_body,
        grid=(grid,),
        in_specs=[
            pl.BlockSpec((N_CORES, RB, K), lambda i: (0, i, 0)),
            pl.BlockSpec((N_CORES, RB, 1), lambda i: (0, i, 0)),
            pl.BlockSpec((RB, 1), lambda i: (i, 0)),
            pl.BlockSpec((1, K), lambda i: (0, 0)),
        ],
        out_specs=pl.BlockSpec((1, 1), lambda i: (0, 0)),
        out_shape=jax.ShapeDtypeStruct((1, 1), f32),
        scratch_shapes=[pltpu.VMEM((1, K), f32)],
        name="hg_loss_tc",
    )(s_p, de_p3, w_pad2, f_dv_f)

    return jnp.reshape(loss2d, ())


# speedup vs baseline: 174.9910x; 1.4854x over previous
import jax
import jax.numpy as jnp
from jax import lax
from jax.experimental import pallas as pl
from jax.experimental.pallas import tpu as pltpu
from jax.experimental.pallas import tpu_sc as plsc

N_CORES = 2
N_SUBCORES = 16
N_WORKERS = N_CORES * N_SUBCORES
N_PAD = 100352
SLICE = N_PAD // N_SUBCORES
WROWS = N_PAD // N_WORKERS
BLK = 448
CH_A = 4000
CH_C = 1000
K = 8


def _rsqrt16(x):
    i = plsc.bitcast(x, jnp.int32)
    y = plsc.bitcast(jnp.full((16,), 0x5F3759DF, jnp.int32) - (i >> 1),
                     jnp.float32)
    for _ in range(3):
        y = y * (1.5 - 0.5 * x * y * y)
    return y


def _degrees_body(hi, w_hbm, z1_hbm, ones_hbm, dv_out, de_out,
                  in_a, ie_a, in_b, ie_b, wv_a, wv_b, ones_v,
                  w_sp, dv_sp, de_sp,
                  sem_la, sem_lb, sem_g, sem_sa, sem_sb):
    cid = lax.axis_index("c")
    sid = lax.axis_index("s")
    wid = cid * N_SUBCORES + sid
    soff = sid * SLICE
    pltpu.sync_copy(w_hbm.at[pl.ds(soff, SLICE)], w_sp.at[pl.ds(soff, SLICE)])
    pltpu.sync_copy(z1_hbm, dv_sp.at[pl.ds(soff, SLICE)])
    pltpu.sync_copy(z1_hbm, de_sp.at[pl.ds(soff, SLICE)])
    pltpu.sync_copy(ones_hbm, ones_v)
    plsc.subcore_barrier()

    n_pairs = hi.shape[0] // 2
    per_w = n_pairs // N_WORKERS
    nit2 = per_w // (2 * CH_A)
    base = wid * per_w

    def loads(i, idx_n, idx_e, sem):
        pltpu.async_copy(hi.at[pl.ds(base + i * CH_A, CH_A)], idx_n, sem)
        pltpu.async_copy(hi.at[pl.ds(n_pairs + base + i * CH_A, CH_A)],
                         idx_e, sem)

    def wait_loads(idx_n, idx_e, sem):
        pltpu.make_async_copy(hi.at[pl.ds(base, CH_A)], idx_n, sem).wait()
        pltpu.make_async_copy(hi.at[pl.ds(base, CH_A)], idx_e, sem).wait()

    def scatters(idx_n, idx_e, wv, sem):
        pltpu.async_copy(wv, dv_sp.at[idx_n], sem, add=True)
        pltpu.async_copy(ones_v, de_sp.at[idx_e], sem, add=True)

    def wait_scatters(idx_n, idx_e, wv, sem):
        pltpu.make_async_copy(wv, dv_sp.at[idx_n], sem).wait()
        pltpu.make_async_copy(ones_v, de_sp.at[idx_e], sem).wait()

    loads(0, in_a, ie_a, sem_la)

    def body(j, carry):
        @pl.when(j > 0)
        def _():
            wait_scatters(in_b, ie_b, wv_b, sem_sb)

        wait_loads(in_a, ie_a, sem_la)
        ga = pltpu.async_copy(w_sp.at[ie_a], wv_a, sem_g)
        loads(2 * j + 1, in_b, ie_b, sem_lb)
        ga.wait()
        scatters(in_a, ie_a, wv_a, sem_sa)
        wait_loads(in_b, ie_b, sem_lb)
        pltpu.async_copy(w_sp.at[ie_b], wv_b, sem_g).wait()
        wait_scatters(in_a, ie_a, wv_a, sem_sa)

        @pl.when(j < nit2 - 1)
        def _():
            loads(2 * j + 2, in_a, ie_a, sem_la)

        scatters(in_b, ie_b, wv_b, sem_sb)
        return carry

    lax.fori_loop(0, nit2, body, 0)
    wait_scatters(in_b, ie_b, wv_b, sem_sb)
    plsc.subcore_barrier()
    pltpu.sync_copy(dv_sp.at[pl.ds(soff, SLICE)],
                    dv_out.at[pl.ds(cid * N_PAD + soff, SLICE)])
    pltpu.sync_copy(de_sp.at[pl.ds(soff, SLICE)],
                    de_out.at[pl.ds(cid * N_PAD + soff, SLICE)])


def _edge_sums_body(hi, dv_hbm, z_hbm, z8_hbm, s_out,
                    dv0_v, dv1_v, rs_v, z_blk,
                    in_a, ie_a, in_b, ie_b, rows_a, rows_b,
                    nz_sp, s_sp,
                    sem_p, sem_la, sem_lb, sem_g, sem_sa, sem_sb):
    cid = lax.axis_index("c")
    sid = lax.axis_index("s")
    wid = cid * N_SUBCORES + sid
    soff = sid * SLICE
    pltpu.sync_copy(z8_hbm, s_sp.at[pl.ds(soff, SLICE)])

    iota = lax.iota(jnp.int32, 16)
    sh3 = iota >> 3
    col = iota & 7

    def blk_body(b, carry):
        row = soff + b * BLK
        pltpu.async_copy(dv_hbm.at[pl.ds(row, BLK)], dv0_v, sem_p)
        pltpu.async_copy(dv_hbm.at[pl.ds(N_PAD + row, BLK)], dv1_v, sem_p)
        pltpu.async_copy(z_hbm.at[pl.ds(row, BLK)], z_blk, sem_p)
        pltpu.make_async_copy(dv_hbm.at[pl.ds(row, BLK)], dv0_v, sem_p).wait()
        pltpu.make_async_copy(dv_hbm.at[pl.ds(row, BLK)], dv1_v, sem_p).wait()
        pltpu.make_async_copy(z_hbm.at[pl.ds(row, BLK)], z_blk, sem_p).wait()

        def rv(r, c2):
            rr = r * 16
            x = dv0_v[pl.ds(rr, 16)] + dv1_v[pl.ds(rr, 16)]
            x = jnp.where(x == 0.0, 1.0, x)
            rs_v[pl.ds(rr, 16)] = _rsqrt16(x)
            return c2

        lax.fori_loop(0, BLK // 16, rv, 0)

        def zv(v, c2):
            for u in range(4):
                vv = (4 * v + u) * 16
                ridx = sh3 + (vv >> 3)
                r16 = plsc.load_gather(rs_v, [ridx])
                z16 = plsc.load_gather(z_blk, [ridx, col])
                plsc.store_scatter(z_blk, [ridx, col], r16 * z16)
            return c2

        lax.fori_loop(0, BLK * K // 64, zv, 0)
        pltpu.sync_copy(z_blk, nz_sp.at[pl.ds(row, BLK)])
        return carry

    lax.fori_loop(0, SLICE // BLK, blk_body, 0)
    plsc.subcore_barrier()

    n_pairs = hi.shape[0] // 2
    per_w = n_pairs // N_WORKERS
    nit2 = per_w // (2 * CH_C)
    base = wid * per_w

    def loads(i, idx_n, idx_e, sem):
        pltpu.async_copy(hi.at[pl.ds(base + i * CH_C, CH_C)], idx_n, sem)
        pltpu.async_copy(hi.at[pl.ds(n_pairs + base + i * CH_C, CH_C)],
                         idx_e, sem)

    def wait_loads(idx_n, idx_e, sem):
        pltpu.make_async_copy(hi.at[pl.ds(base, CH_C)], idx_n, sem).wait()
        pltpu.make_async_copy(hi.at[pl.ds(base, CH_C)], idx_e, sem).wait()

    loads(0, in_a, ie_a, sem_la)

    def body(j, carry):
        @pl.when(j > 0)
        def _():
            pltpu.make_async_copy(rows_b, s_sp.at[ie_b], sem_sb).wait()

        wait_loads(in_a, ie_a, sem_la)
        ga = pltpu.async_copy(nz_sp.at[in_a], rows_a, sem_g)
        loads(2 * j + 1, in_b, ie_b, sem_lb)
        ga.wait()
        pltpu.async_copy(rows_a, s_sp.at[ie_a], sem_sa, add=True)
        wait_loads(in_b, ie_b, sem_lb)
        pltpu.async_copy(nz_sp.at[in_b], rows_b, sem_g).wait()
        pltpu.make_async_copy(rows_a, s_sp.at[ie_a], sem_sa).wait()

        @pl.when(j < nit2 - 1)
        def _():
            loads(2 * j + 2, in_a, ie_a, sem_la)

        pltpu.async_copy(rows_b, s_sp.at[ie_b], sem_sb, add=True)
        return carry

    lax.fori_loop(0, nit2, body, 0)
    pltpu.make_async_copy(rows_b, s_sp.at[ie_b], sem_sb).wait()
    plsc.subcore_barrier()
    pltpu.sync_copy(s_sp.at[pl.ds(soff, SLICE)],
                    s_out.at[cid, pl.ds(soff, SLICE)])


def _reduce_body(s_hbm, dv_hbm, de_hbm, w_hbm, z_hbm, th_out, f_out,
                 s0_blk, s1_blk, z_blk, dv0_v, dv1_v, de0_v, de1_v, wv_v,
                 wde_v, dvc_v, th_acc, f_acc, sem_p):
    cid = lax.axis_index("c")
    sid = lax.axis_index("s")
    wid = cid * N_SUBCORES + sid
    wbase = wid * WROWS

    iota = lax.iota(jnp.int32, 16)
    sh3 = iota >> 3
    col = iota & 7
    zero16 = jnp.zeros((16,), jnp.float32)
    th_acc[...] = zero16
    f_acc[...] = zero16

    def blk_body(b, carry):
        row = wbase + b * BLK
        pltpu.async_copy(s_hbm.at[0, pl.ds(row, BLK)], s0_blk, sem_p)
        pltpu.async_copy(s_hbm.at[1, pl.ds(row, BLK)], s1_blk, sem_p)
        pltpu.async_copy(z_hbm.at[pl.ds(row, BLK)], z_blk, sem_p)
        pltpu.async_copy(dv_hbm.at[pl.ds(row, BLK)], dv0_v, sem_p)
        pltpu.async_copy(dv_hbm.at[pl.ds(N_PAD + row, BLK)], dv1_v, sem_p)
        pltpu.async_copy(de_hbm.at[pl.ds(row, BLK)], de0_v, sem_p)
        pltpu.async_copy(de_hbm.at[pl.ds(N_PAD + row, BLK)], de1_v, sem_p)
        pltpu.async_copy(w_hbm.at[pl.ds(row, BLK)], wv_v, sem_p)
        pltpu.make_async_copy(s_hbm.at[0, pl.ds(row, BLK)], s0_blk, sem_p).wait()
        pltpu.make_async_copy(s_hbm.at[1, pl.ds(row, BLK)], s1_blk, sem_p).wait()
        pltpu.make_async_copy(z_hbm.at[pl.ds(row, BLK)], z_blk, sem_p).wait()
        pltpu.make_async_copy(dv_hbm.at[pl.ds(row, BLK)], dv0_v, sem_p).wait()
        pltpu.make_async_copy(dv_hbm.at[pl.ds(row, BLK)], dv1_v, sem_p).wait()
        pltpu.make_async_copy(de_hbm.at[pl.ds(row, BLK)], de0_v, sem_p).wait()
        pltpu.make_async_copy(de_hbm.at[pl.ds(row, BLK)], de1_v, sem_p).wait()
        pltpu.make_async_copy(w_hbm.at[pl.ds(row, BLK)], wv_v, sem_p).wait()

        def rv(r, c2):
            rr = r * 16
            de = de0_v[pl.ds(rr, 16)] + de1_v[pl.ds(rr, 16)]
            de = jnp.where(de == 0.0, 1.0, de)
            wde_v[pl.ds(rr, 16)] = wv_v[pl.ds(rr, 16)] / de
            dv = dv0_v[pl.ds(rr, 16)] + dv1_v[pl.ds(rr, 16)]
            dvc_v[pl.ds(rr, 16)] = jnp.where(dv == 0.0, 1.0, dv)
            return c2

        lax.fori_loop(0, BLK // 16, rv, 0)

        def zv(v, c2):
            th = th_acc[...]
            f = f_acc[...]
            for u in range(4):
                vv = (4 * v + u) * 16
                ridx = sh3 + (vv >> 3)
                s = (plsc.load_gather(s0_blk, [ridx, col])
                     + plsc.load_gather(s1_blk, [ridx, col]))
                zz = plsc.load_gather(z_blk, [ridx, col])
                wd = plsc.load_gather(wde_v, [ridx])
                dc = plsc.load_gather(dvc_v, [ridx])
                th = th + wd * s * s
                f = f + zz * zz * dc
            th_acc[...] = th
            f_acc[...] = f
            return c2

        lax.fori_loop(0, BLK * K // 64, zv, 0)
        return carry

    lax.fori_loop(0, WROWS // BLK, blk_body, 0)
    pltpu.sync_copy(th_acc, th_out.at[cid, pl.ds(sid * 16, 16)])
    pltpu.sync_copy(f_acc, f_out.at[cid, pl.ds(sid * 16, 16)])


def kernel(Z, hyperedge_index, num_nodes, hyperedge_weight):
    del num_nodes
    n = Z.shape[0]
    f32 = jnp.float32

    hi_flat = jnp.reshape(hyperedge_index, (-1,))
    z_pad = jnp.pad(Z.astype(f32), ((0, N_PAD - n), (0, 0)))
    w_pad = jnp.pad(hyperedge_weight.astype(f32), (0, N_PAD - n))
    zeros1 = jnp.zeros((SLICE,), f32)
    zeros8 = jnp.zeros((SLICE, K), f32)
    ones_ch = jnp.ones((CH_A,), f32)

    mesh = plsc.VectorSubcoreMesh(core_axis_name="c", subcore_axis_name="s")
    sc_params = pltpu.CompilerParams(use_tc_tiling_on_sc=False,
                                     needs_layout_passes=False)

    degrees = pl.kernel(
        _degrees_body,
        out_type=(
            jax.ShapeDtypeStruct((N_CORES * N_PAD,), f32),
            jax.ShapeDtypeStruct((N_CORES * N_PAD,), f32),
        ),
        mesh=mesh,
        scratch_types=(
            pltpu.VMEM((CH_A,), jnp.int32),
            pltpu.VMEM((CH_A,), jnp.int32),
            pltpu.VMEM((CH_A,), jnp.int32),
            pltpu.VMEM((CH_A,), jnp.int32),
            pltpu.VMEM((CH_A,), f32),
            pltpu.VMEM((CH_A,), f32),
            pltpu.VMEM((CH_A,), f32),
            pltpu.VMEM_SHARED((N_PAD,), f32),
            pltpu.VMEM_SHARED((N_PAD,), f32),
            pltpu.VMEM_SHARED((N_PAD,), f32),
            pltpu.SemaphoreType.DMA,
            pltpu.SemaphoreType.DMA,
            pltpu.SemaphoreType.DMA,
            pltpu.SemaphoreType.DMA,
            pltpu.SemaphoreType.DMA,
        ),
        compiler_params=sc_params,
        name="hg_degrees_sc",
    )
    dv_p, de_p = degrees(hi_flat, w_pad, zeros1, ones_ch)

    edge_sums = pl.kernel(
        _edge_sums_body,
        out_type=jax.ShapeDtypeStruct((N_CORES, N_PAD, K), f32),
        mesh=mesh,
        scratch_types=(
            pltpu.VMEM((BLK,), f32),
            pltpu.VMEM((BLK,), f32),
            pltpu.VMEM((BLK,), f32),
            pltpu.VMEM((BLK, K), f32),
            pltpu.VMEM((CH_C,), jnp.int32),
            pltpu.VMEM((CH_C,), jnp.int32),
            pltpu.VMEM((CH_C,), jnp.int32),
            pltpu.VMEM((CH_C,), jnp.int32),
            pltpu.VMEM((CH_C, K), f32),
            pltpu.VMEM((CH_C, K), f32),
            pltpu.VMEM_SHARED((N_PAD, K), f32),
            pltpu.VMEM_SHARED((N_PAD, K), f32),
            pltpu.SemaphoreType.DMA,
            pltpu.SemaphoreType.DMA,
            pltpu.SemaphoreType.DMA,
            pltpu.SemaphoreType.DMA,
            pltpu.SemaphoreType.DMA,
            pltpu.SemaphoreType.DMA,
        ),
        compiler_params=sc_params,
        name="hg_edge_sums_sc",
    )
    s_p = edge_sums(hi_flat, dv_p, z_pad, zeros8)

    reduce_k = pl.kernel(
        _reduce_body,
        out_type=(
            jax.ShapeDtypeStruct((N_CORES, N_SUBCORES * 16), f32),
            jax.ShapeDtypeStruct((N_CORES, N_SUBCORES * 16), f32),
        ),
        mesh=mesh,
        scratch_types=(
            pltpu.VMEM((BLK, K), f32),
            pltpu.VMEM((BLK, K), f32),
            pltpu.VMEM((BLK, K), f32),
            pltpu.VMEM((BLK,), f32),
            pltpu.VMEM((BLK,), f32),
            pltpu.VMEM((BLK,), f32),
            pltpu.VMEM((BLK,), f32),
            pltpu.VMEM((BLK,), f32),
            pltpu.VMEM((BLK,), f32),
            pltpu.VMEM((BLK,), f32),
            pltpu.VMEM((16,), f32),
            pltpu.VMEM((16,), f32),
            pltpu.SemaphoreType.DMA,
        ),
        compiler_params=sc_params,
        name="hg_reduce_sc",
    )
    th_p, f_p = reduce_k(s_p, dv_p, de_p, w_pad, z_pad)

    theta = jnp.sum(jnp.reshape(th_p, (-1, 2, K)), axis=(0, 1))
    f_dv_f = jnp.sum(jnp.reshape(f_p, (-1, 2, K)), axis=(0, 1))
    loss = jnp.mean(1.0 - theta / (f_dv_f + 1e-8))
    return loss.astype(f32)


# speedup vs baseline: 186.2561x; 1.0644x over previous
import jax
import jax.numpy as jnp
from jax import lax
from jax.experimental import pallas as pl
from jax.experimental.pallas import tpu as pltpu
from jax.experimental.pallas import tpu_sc as plsc

N_CORES = 2
N_SUBCORES = 16
N_WORKERS = N_CORES * N_SUBCORES
N_PAD = 100352
SLICE = N_PAD // N_SUBCORES
WROWS = N_PAD // N_WORKERS
BLK = 448
CH_A = 8000
CH_C = 1000
K = 8


def _rsqrt16(x):
    i = plsc.bitcast(x, jnp.int32)
    y = plsc.bitcast(jnp.full((16,), 0x5F3759DF, jnp.int32) - (i >> 1),
                     jnp.float32)
    for _ in range(3):
        y = y * (1.5 - 0.5 * x * y * y)
    return y


def _degrees_body(hi, w_hbm, z1_hbm, ones_hbm, dv_out, de_out,
                  in_a, ie_a, in_b, ie_b, wv_a, wv_b, ones_v,
                  w_sp, dv_sp, de_sp,
                  sem_la, sem_lb, sem_g, sem_sa, sem_sb):
    cid = lax.axis_index("c")
    sid = lax.axis_index("s")
    wid = cid * N_SUBCORES + sid
    soff = sid * SLICE
    pltpu.sync_copy(w_hbm.at[pl.ds(soff, SLICE)], w_sp.at[pl.ds(soff, SLICE)])
    pltpu.sync_copy(z1_hbm, dv_sp.at[pl.ds(soff, SLICE)])
    pltpu.sync_copy(z1_hbm, de_sp.at[pl.ds(soff, SLICE)])
    pltpu.sync_copy(ones_hbm, ones_v)
    plsc.subcore_barrier()

    n_pairs = hi.shape[0] // 2
    per_w = n_pairs // N_WORKERS
    nit2 = per_w // (2 * CH_A)
    base = wid * per_w

    def loads(i, idx_n, idx_e, sem):
        pltpu.async_copy(hi.at[pl.ds(base + i * CH_A, CH_A)], idx_n, sem)
        pltpu.async_copy(hi.at[pl.ds(n_pairs + base + i * CH_A, CH_A)],
                         idx_e, sem)

    def wait_loads(idx_n, idx_e, sem):
        pltpu.make_async_copy(hi.at[pl.ds(base, CH_A)], idx_n, sem).wait()
        pltpu.make_async_copy(hi.at[pl.ds(base, CH_A)], idx_e, sem).wait()

    def scatters(idx_n, idx_e, wv, sem):
        pltpu.async_copy(wv, dv_sp.at[idx_n], sem, add=True)
        pltpu.async_copy(ones_v, de_sp.at[idx_e], sem, add=True)

    def wait_scatters(idx_n, idx_e, wv, sem):
        pltpu.make_async_copy(wv, dv_sp.at[idx_n], sem).wait()
        pltpu.make_async_copy(ones_v, de_sp.at[idx_e], sem).wait()

    loads(0, in_a, ie_a, sem_la)

    def body(j, carry):
        @pl.when(j > 0)
        def _():
            wait_scatters(in_b, ie_b, wv_b, sem_sb)

        wait_loads(in_a, ie_a, sem_la)
        ga = pltpu.async_copy(w_sp.at[ie_a], wv_a, sem_g)
        loads(2 * j + 1, in_b, ie_b, sem_lb)
        ga.wait()
        scatters(in_a, ie_a, wv_a, sem_sa)
        wait_loads(in_b, ie_b, sem_lb)
        pltpu.async_copy(w_sp.at[ie_b], wv_b, sem_g).wait()
        wait_scatters(in_a, ie_a, wv_a, sem_sa)

        @pl.when(j < nit2 - 1)
        def _():
            loads(2 * j + 2, in_a, ie_a, sem_la)

        scatters(in_b, ie_b, wv_b, sem_sb)
        return carry

    lax.fori_loop(0, nit2, body, 0)
    wait_scatters(in_b, ie_b, wv_b, sem_sb)
    plsc.subcore_barrier()
    pltpu.sync_copy(dv_sp.at[pl.ds(soff, SLICE)],
                    dv_out.at[pl.ds(cid * N_PAD + soff, SLICE)])
    pltpu.sync_copy(de_sp.at[pl.ds(soff, SLICE)],
                    de_out.at[pl.ds(cid * N_PAD + soff, SLICE)])


def _edge_sums_body(hi, dv_hbm, z_hbm, z8_hbm, s_out,
                    dv0_v, dv1_v, rs_v, z_blk,
                    in_a, ie_a, in_b, ie_b, rows_a, rows_b,
                    nz_sp, s_sp,
                    sem_p, sem_la, sem_lb, sem_g, sem_sa, sem_sb):
    cid = lax.axis_index("c")
    sid = lax.axis_index("s")
    wid = cid * N_SUBCORES + sid
    soff = sid * SLICE
    n_pairs = hi.shape[0] // 2
    per_w = n_pairs // N_WORKERS
    nit2 = per_w // (2 * CH_C)
    base = wid * per_w

    def loads(i, idx_n, idx_e, sem):
        pltpu.async_copy(hi.at[pl.ds(base + i * CH_C, CH_C)], idx_n, sem)
        pltpu.async_copy(hi.at[pl.ds(n_pairs + base + i * CH_C, CH_C)],
                         idx_e, sem)

    def wait_loads(idx_n, idx_e, sem):
        pltpu.make_async_copy(hi.at[pl.ds(base, CH_C)], idx_n, sem).wait()
        pltpu.make_async_copy(hi.at[pl.ds(base, CH_C)], idx_e, sem).wait()

    loads(0, in_a, ie_a, sem_la)
    pltpu.sync_copy(z8_hbm, s_sp.at[pl.ds(soff, SLICE)])

    iota = lax.iota(jnp.int32, 16)
    sh3 = iota >> 3
    col = iota & 7

    def blk_body(b, carry):
        row = soff + b * BLK
        pltpu.async_copy(dv_hbm.at[pl.ds(row, BLK)], dv0_v, sem_p)
        pltpu.async_copy(dv_hbm.at[pl.ds(N_PAD + row, BLK)], dv1_v, sem_p)
        pltpu.async_copy(z_hbm.at[pl.ds(row, BLK)], z_blk, sem_p)
        pltpu.make_async_copy(dv_hbm.at[pl.ds(row, BLK)], dv0_v, sem_p).wait()
        pltpu.make_async_copy(dv_hbm.at[pl.ds(row, BLK)], dv1_v, sem_p).wait()
        pltpu.make_async_copy(z_hbm.at[pl.ds(row, BLK)], z_blk, sem_p).wait()

        def rv(r, c2):
            rr = r * 16
            x = dv0_v[pl.ds(rr, 16)] + dv1_v[pl.ds(rr, 16)]
            x = jnp.where(x == 0.0, 1.0, x)
            rs_v[pl.ds(rr, 16)] = _rsqrt16(x)
            return c2

        lax.fori_loop(0, BLK // 16, rv, 0)

        def zv(v, c2):
            for u in range(4):
                vv = (4 * v + u) * 16
                ridx = sh3 + (vv >> 3)
                r16 = plsc.load_gather(rs_v, [ridx])
                z16 = plsc.load_gather(z_blk, [ridx, col])
                plsc.store_scatter(z_blk, [ridx, col], r16 * z16)
            return c2

        lax.fori_loop(0, BLK * K // 64, zv, 0)
        pltpu.sync_copy(z_blk, nz_sp.at[pl.ds(row, BLK)])
        return carry

    lax.fori_loop(0, SLICE // BLK, blk_body, 0)
    plsc.subcore_barrier()

    def body(j, carry):
        @pl.when(j > 0)
        def _():
            pltpu.make_async_copy(rows_b, s_sp.at[ie_b], sem_sb).wait()

        wait_loads(in_a, ie_a, sem_la)
        ga = pltpu.async_copy(nz_sp.at[in_a], rows_a, sem_g)
        loads(2 * j + 1, in_b, ie_b, sem_lb)
        ga.wait()
        pltpu.async_copy(rows_a, s_sp.at[ie_a], sem_sa, add=True)
        wait_loads(in_b, ie_b, sem_lb)
        pltpu.async_copy(nz_sp.at[in_b], rows_b, sem_g).wait()
        pltpu.make_async_copy(rows_a, s_sp.at[ie_a], sem_sa).wait()

        @pl.when(j < nit2 - 1)
        def _():
            loads(2 * j + 2, in_a, ie_a, sem_la)

        pltpu.async_copy(rows_b, s_sp.at[ie_b], sem_sb, add=True)
        return carry

    lax.fori_loop(0, nit2, body, 0)
    pltpu.make_async_copy(rows_b, s_sp.at[ie_b], sem_sb).wait()
    plsc.subcore_barrier()
    pltpu.sync_copy(s_sp.at[pl.ds(soff, SLICE)],
                    s_out.at[cid, pl.ds(soff, SLICE)])


def _reduce_body(s_hbm, dv_hbm, de_hbm, w_hbm, z_hbm, th_out, f_out,
                 s0_blk, s1_blk, z_blk, dv0_v, dv1_v, de0_v, de1_v, wv_v,
                 wde_v, dvc_v, th_acc, f_acc, sem_p):
    cid = lax.axis_index("c")
    sid = lax.axis_index("s")
    wid = cid * N_SUBCORES + sid
    wbase = wid * WROWS

    iota = lax.iota(jnp.int32, 16)
    sh3 = iota >> 3
    col = iota & 7
    zero16 = jnp.zeros((16,), jnp.float32)
    th_acc[...] = zero16
    f_acc[...] = zero16

    def blk_body(b, carry):
        row = wbase + b * BLK
        pltpu.async_copy(s_hbm.at[0, pl.ds(row, BLK)], s0_blk, sem_p)
        pltpu.async_copy(s_hbm.at[1, pl.ds(row, BLK)], s1_blk, sem_p)
        pltpu.async_copy(z_hbm.at[pl.ds(row, BLK)], z_blk, sem_p)
        pltpu.async_copy(dv_hbm.at[pl.ds(row, BLK)], dv0_v, sem_p)
        pltpu.async_copy(dv_hbm.at[pl.ds(N_PAD + row, BLK)], dv1_v, sem_p)
        pltpu.async_copy(de_hbm.at[pl.ds(row, BLK)], de0_v, sem_p)
        pltpu.async_copy(de_hbm.at[pl.ds(N_PAD + row, BLK)], de1_v, sem_p)
        pltpu.async_copy(w_hbm.at[pl.ds(row, BLK)], wv_v, sem_p)
        pltpu.make_async_copy(s_hbm.at[0, pl.ds(row, BLK)], s0_blk, sem_p).wait()
        pltpu.make_async_copy(s_hbm.at[1, pl.ds(row, BLK)], s1_blk, sem_p).wait()
        pltpu.make_async_copy(z_hbm.at[pl.ds(row, BLK)], z_blk, sem_p).wait()
        pltpu.make_async_copy(dv_hbm.at[pl.ds(row, BLK)], dv0_v, sem_p).wait()
        pltpu.make_async_copy(dv_hbm.at[pl.ds(row, BLK)], dv1_v, sem_p).wait()
        pltpu.make_async_copy(de_hbm.at[pl.ds(row, BLK)], de0_v, sem_p).wait()
        pltpu.make_async_copy(de_hbm.at[pl.ds(row, BLK)], de1_v, sem_p).wait()
        pltpu.make_async_copy(w_hbm.at[pl.ds(row, BLK)], wv_v, sem_p).wait()

        def rv(r, c2):
            rr = r * 16
            de = de0_v[pl.ds(rr, 16)] + de1_v[pl.ds(rr, 16)]
            de = jnp.where(de == 0.0, 1.0, de)
            wde_v[pl.ds(rr, 16)] = wv_v[pl.ds(rr, 16)] / de
            dv = dv0_v[pl.ds(rr, 16)] + dv1_v[pl.ds(rr, 16)]
            dvc_v[pl.ds(rr, 16)] = jnp.where(dv == 0.0, 1.0, dv)
            return c2

        lax.fori_loop(0, BLK // 16, rv, 0)

        def zv(v, c2):
            th = th_acc[...]
            f = f_acc[...]
            for u in range(4):
                vv = (4 * v + u) * 16
                ridx = sh3 + (vv >> 3)
                s = (plsc.load_gather(s0_blk, [ridx, col])
                     + plsc.load_gather(s1_blk, [ridx, col]))
                zz = plsc.load_gather(z_blk, [ridx, col])
                wd = plsc.load_gather(wde_v, [ridx])
                dc = plsc.load_gather(dvc_v, [ridx])
                th = th + wd * s * s
                f = f + zz * zz * dc
            th_acc[...] = th
            f_acc[...] = f
            return c2

        lax.fori_loop(0, BLK * K // 64, zv, 0)
        return carry

    lax.fori_loop(0, WROWS // BLK, blk_body, 0)
    pltpu.sync_copy(th_acc, th_out.at[cid, pl.ds(sid * 16, 16)])
    pltpu.sync_copy(f_acc, f_out.at[cid, pl.ds(sid * 16, 16)])


def kernel(Z, hyperedge_index, num_nodes, hyperedge_weight):
    del num_nodes
    n = Z.shape[0]
    f32 = jnp.float32

    hi_flat = jnp.reshape(hyperedge_index, (-1,))
    z_pad = jnp.pad(Z.astype(f32), ((0, N_PAD - n), (0, 0)))
    w_pad = jnp.pad(hyperedge_weight.astype(f32), (0, N_PAD - n))
    zeros1 = jnp.zeros((SLICE,), f32)
    zeros8 = jnp.zeros((SLICE, K), f32)
    ones_ch = jnp.ones((CH_A,), f32)

    mesh = plsc.VectorSubcoreMesh(core_axis_name="c", subcore_axis_name="s")
    sc_params = pltpu.CompilerParams(use_tc_tiling_on_sc=False,
                                     needs_layout_passes=False)

    degrees = pl.kernel(
        _degrees_body,
        out_type=(
            jax.ShapeDtypeStruct((N_CORES * N_PAD,), f32),
            jax.ShapeDtypeStruct((N_CORES * N_PAD,), f32),
        ),
        mesh=mesh,
        scratch_types=(
            pltpu.VMEM((CH_A,), jnp.int32),
            pltpu.VMEM((CH_A,), jnp.int32),
            pltpu.VMEM((CH_A,), jnp.int32),
            pltpu.VMEM((CH_A,), jnp.int32),
            pltpu.VMEM((CH_A,), f32),
            pltpu.VMEM((CH_A,), f32),
            pltpu.VMEM((CH_A,), f32),
            pltpu.VMEM_SHARED((N_PAD,), f32),
            pltpu.VMEM_SHARED((N_PAD,), f32),
            pltpu.VMEM_SHARED((N_PAD,), f32),
            pltpu.SemaphoreType.DMA,
            pltpu.SemaphoreType.DMA,
            pltpu.SemaphoreType.DMA,
            pltpu.SemaphoreType.DMA,
            pltpu.SemaphoreType.DMA,
        ),
        compiler_params=sc_params,
        name="hg_degrees_sc",
    )
    dv_p, de_p = degrees(hi_flat, w_pad, zeros1, ones_ch)

    edge_sums = pl.kernel(
        _edge_sums_body,
        out_type=jax.ShapeDtypeStruct((N_CORES, N_PAD, K), f32),
        mesh=mesh,
        scratch_types=(
            pltpu.VMEM((BLK,), f32),
            pltpu.VMEM((BLK,), f32),
            pltpu.VMEM((BLK,), f32),
            pltpu.VMEM((BLK, K), f32),
            pltpu.VMEM((CH_C,), jnp.int32),
            pltpu.VMEM((CH_C,), jnp.int32),
            pltpu.VMEM((CH_C,), jnp.int32),
            pltpu.VMEM((CH_C,), jnp.int32),
            pltpu.VMEM((CH_C, K), f32),
            pltpu.VMEM((CH_C, K), f32),
            pltpu.VMEM_SHARED((N_PAD, K), f32),
            pltpu.VMEM_SHARED((N_PAD, K), f32),
            pltpu.SemaphoreType.DMA,
            pltpu.SemaphoreType.DMA,
            pltpu.SemaphoreType.DMA,
            pltpu.SemaphoreType.DMA,
            pltpu.SemaphoreType.DMA,
            pltpu.SemaphoreType.DMA,
        ),
        compiler_params=sc_params,
        name="hg_edge_sums_sc",
    )
    s_p = edge_sums(hi_flat, dv_p, z_pad, zeros8)

    reduce_k = pl.kernel(
        _reduce_body,
        out_type=(
            jax.ShapeDtypeStruct((N_CORES, N_SUBCORES * 16), f32),
            jax.ShapeDtypeStruct((N_CORES, N_SUBCORES * 16), f32),
        ),
        mesh=mesh,
        scratch_types=(
            pltpu.VMEM((BLK, K), f32),
            pltpu.VMEM((BLK, K), f32),
            pltpu.VMEM((BLK, K), f32),
            pltpu.VMEM((BLK,), f32),
            pltpu.VMEM((BLK,), f32),
            pltpu.VMEM((BLK,), f32),
            pltpu.VMEM((BLK,), f32),
            pltpu.VMEM((BLK,), f32),
            pltpu.VMEM((BLK,), f32),
            pltpu.VMEM((BLK,), f32),
            pltpu.VMEM((16,), f32),
            pltpu.VMEM((16,), f32),
            pltpu.SemaphoreType.DMA,
        ),
        compiler_params=sc_params,
        name="hg_reduce_sc",
    )
    th_p, f_p = reduce_k(s_p, dv_p, de_p, w_pad, z_pad)

    theta = jnp.sum(jnp.reshape(th_p, (-1, 2, K)), axis=(0, 1))
    f_dv_f = jnp.sum(jnp.reshape(f_p, (-1, 2, K)), axis=(0, 1))
    loss = jnp.mean(1.0 - theta / (f_dv_f + 1e-8))
    return loss.astype(f32)
